# Initial kernel scaffold; baseline (speedup 1.0000x reference)
import jax
import jax.numpy as jnp
from jax.experimental import pallas as pl


def kernel(input_ids, embed_tokens, embed_positions):
    raise NotImplementedError("write your pallas kernel here")


# speedup vs baseline: 1.4359x; 1.4359x over previous
import functools

import jax
import jax.numpy as jnp
from jax import lax
from jax.experimental import pallas as pl
from jax.experimental.pallas import tpu as pltpu
from jax.experimental.pallas import tpu_sc as plsc

_PAD = 1
_B, _S, _D = 4, 2048, 1024
_NC, _NS, _L = 2, 16, 16
_NW = _NC * _NS
_TPW = (_B * _S) // _NW
_WPR = _NW // _B
_G = 16
_NCH = _TPW // _G

_mesh = plsc.VectorSubcoreMesh(core_axis_name="c", subcore_axis_name="s")


@functools.partial(
    pl.kernel,
    out_type=(
        jax.ShapeDtypeStruct((_B * _S, _D), jnp.float32),
        jax.ShapeDtypeStruct((_B * _S,), jnp.int32),
    ),
    mesh=_mesh,
    scratch_types=[
        pltpu.VMEM((_S,), jnp.int32),
        pltpu.VMEM((_TPW,), jnp.int32),
        pltpu.VMEM((_TPW,), jnp.int32),
        pltpu.VMEM((_G, _D), jnp.float32),
        pltpu.VMEM((_G, _D), jnp.float32),
        pltpu.VMEM((_G, _D), jnp.float32),
        pltpu.VMEM((_G, _D), jnp.float32),
        pltpu.SemaphoreType.DMA,
        pltpu.SemaphoreType.DMA,
        pltpu.SemaphoreType.DMA,
        pltpu.SemaphoreType.DMA,
        pltpu.SemaphoreType.DMA,
        pltpu.SemaphoreType.DMA,
    ],
    compiler_params=pltpu.CompilerParams(needs_layout_passes=False),
)
def _embed_kernel(ids_hbm, tok_hbm, pos_hbm, out_hbm, attn_hbm,
                  row_v, pid_v, attn_v, tok0, tok1, pos0, pos1,
                  st0, st1, sp0, sp1, so0, so1):
    wid = lax.axis_index("s") * _NC + lax.axis_index("c")
    brow = wid // _WPR
    o = (wid % _WPR) * _TPW

    toks = (tok0, tok1)
    poss = (pos0, pos1)
    semt = (st0, st1)
    semp = (sp0, sp1)
    semo = (so0, so1)

    pltpu.sync_copy(ids_hbm.at[pl.ds(brow * _S, _S)], row_v)

    def _pc_body(i, acc):
        ids16 = row_v[pl.ds(i * _L, _L)]
        return acc + jnp.where(ids16 != _PAD, 1, 0)

    acc = lax.fori_loop(0, o // _L, _pc_body, jnp.zeros((_L,), jnp.int32))
    carry0 = jnp.sum(acc)

    def _pos_body(k, carry):
        ids16 = row_v[pl.ds(o + k * _L, _L)]
        attn16 = jnp.where(ids16 != _PAD, 1, 0)
        cum = plsc.cumsum(attn16)
        pos16 = jnp.maximum(carry + cum - 1, 0)
        pid_v[pl.ds(k * _L, _L)] = pos16
        attn_v[pl.ds(k * _L, _L)] = attn16
        return carry + jnp.sum(attn16)

    lax.fori_loop(0, _TPW // _L, _pos_body, carry0)

    pltpu.sync_copy(attn_v, attn_hbm.at[pl.ds(wid * _TPW, _TPW)])

    def _start_gather(ch, bb):
        cbase = ch * _G
        ht = pltpu.async_copy(
            tok_hbm.at[row_v.at[pl.ds(o + cbase, _G)]], toks[bb], semt[bb])
        hp = pltpu.async_copy(
            pos_hbm.at[pid_v.at[pl.ds(cbase, _G)]], poss[bb], semp[bb])
        return ht, hp

    def _accum(bb):
        tr, pr = toks[bb], poss[bb]

        @plsc.parallel_loop(0, _G)
        def _(r):
            for j in range(_D // _L):
                x = pr[r, pl.ds(j * _L, _L)]
                plsc.addupdate(tr.at[r, pl.ds(j * _L, _L)], x)

    gat = [None, None]
    out = [None, None]
    gat[0] = _start_gather(0, 0)
    for ch in range(_NCH):
        bb = ch % 2
        nb = (ch + 1) % 2
        if ch + 1 < _NCH:
            if out[nb] is not None:
                out[nb].wait()
                out[nb] = None
            gat[nb] = _start_gather(ch + 1, nb)
        ht, hp = gat[bb]
        ht.wait()
        hp.wait()
        _accum(bb)
        out[bb] = pltpu.async_copy(
            toks[bb],
            out_hbm.at[pl.ds(brow * _S + o + ch * _G, _G), :],
            semo[bb])
    for h in out:
        if h is not None:
            h.wait()


def kernel(input_ids, embed_tokens, embed_positions):
    ids_flat = input_ids.reshape(-1)
    hidden, attn = _embed_kernel(ids_flat, embed_tokens, embed_positions)
    return hidden.reshape(_B, _S, _D), attn.reshape(_B, _S)


# speedup vs baseline: 1.4928x; 1.0396x over previous
import functools

import jax
import jax.numpy as jnp
from jax import lax
from jax.experimental import pallas as pl
from jax.experimental.pallas import tpu as pltpu
from jax.experimental.pallas import tpu_sc as plsc

_PAD = 1
_B, _S, _D = 4, 2048, 1024
_NC, _NS, _L = 2, 16, 16
_NW = _NC * _NS
_TPW = (_B * _S) // _NW
_WPR = _NW // _B
_G = 16
_NCH = _TPW // _G

_mesh = plsc.VectorSubcoreMesh(core_axis_name="c", subcore_axis_name="s")


@functools.partial(
    pl.kernel,
    out_type=(
        jax.ShapeDtypeStruct((_B * _S, _D), jnp.float32),
        jax.ShapeDtypeStruct((_B * _S,), jnp.int32),
    ),
    mesh=_mesh,
    scratch_types=[
        pltpu.VMEM((_S,), jnp.int32),
        pltpu.VMEM((_TPW,), jnp.int32),
        pltpu.VMEM((_TPW,), jnp.int32),
        pltpu.VMEM((_G, _D), jnp.float32),
        pltpu.VMEM((_G, _D), jnp.float32),
        pltpu.VMEM((_G, _D), jnp.float32),
        pltpu.VMEM((_G, _D), jnp.float32),
        pltpu.VMEM((_G, _D), jnp.float32),
        pltpu.VMEM((_G, _D), jnp.float32),
        pltpu.SemaphoreType.DMA,
        pltpu.SemaphoreType.DMA,
        pltpu.SemaphoreType.DMA,
        pltpu.SemaphoreType.DMA,
        pltpu.SemaphoreType.DMA,
        pltpu.SemaphoreType.DMA,
        pltpu.SemaphoreType.DMA,
        pltpu.SemaphoreType.DMA,
        pltpu.SemaphoreType.DMA,
    ],
    compiler_params=pltpu.CompilerParams(needs_layout_passes=False),
)
def _embed_kernel(ids_hbm, tok_hbm, pos_hbm, out_hbm, attn_hbm,
                  row_v, pid_v, attn_v, tok0, tok1, tok2, pos0, pos1, pos2,
                  st0, st1, st2, sp0, sp1, sp2, so0, so1, so2):
    wid = lax.axis_index("s") * _NC + lax.axis_index("c")
    brow = wid // _WPR
    o = (wid % _WPR) * _TPW

    toks = (tok0, tok1, tok2)
    poss = (pos0, pos1, pos2)
    semt = (st0, st1, st2)
    semp = (sp0, sp1, sp2)
    semo = (so0, so1, so2)

    pltpu.sync_copy(ids_hbm.at[pl.ds(brow * _S, _S)], row_v)

    def _pc_body(i, acc):
        ids16 = row_v[pl.ds(i * _L, _L)]
        return acc + jnp.where(ids16 != _PAD, 1, 0)

    acc = lax.fori_loop(0, o // _L, _pc_body, jnp.zeros((_L,), jnp.int32))
    carry0 = jnp.sum(acc)

    def _pos_body(k, carry):
        ids16 = row_v[pl.ds(o + k * _L, _L)]
        attn16 = jnp.where(ids16 != _PAD, 1, 0)
        cum = plsc.cumsum(attn16)
        pos16 = jnp.maximum(carry + cum - 1, 0)
        pid_v[pl.ds(k * _L, _L)] = pos16
        attn_v[pl.ds(k * _L, _L)] = attn16
        return carry + jnp.sum(attn16)

    lax.fori_loop(0, _TPW // _L, _pos_body, carry0)

    def _start_gather(ch, bb):
        cbase = ch * _G
        ht = pltpu.async_copy(
            tok_hbm.at[row_v.at[pl.ds(o + cbase, _G)]], toks[bb], semt[bb])
        hp = pltpu.async_copy(
            pos_hbm.at[pid_v.at[pl.ds(cbase, _G)]], poss[bb], semp[bb])
        return ht, hp

    def _accum(bb):
        tr, pr = toks[bb], poss[bb]

        @plsc.parallel_loop(0, _G)
        def _(r):
            for j in range(_D // _L):
                x = pr[r, pl.ds(j * _L, _L)]
                plsc.addupdate(tr.at[r, pl.ds(j * _L, _L)], x)

    _DEPTH = 3
    gat = [None] * _DEPTH
    out = [None] * _DEPTH
    for pf in range(_DEPTH - 1):
        gat[pf % _DEPTH] = _start_gather(pf, pf % _DEPTH)
    pltpu.sync_copy(attn_v, attn_hbm.at[pl.ds(wid * _TPW, _TPW)])
    for ch in range(_NCH):
        bb = ch % _DEPTH
        pf = ch + _DEPTH - 1
        if pf < _NCH:
            pb = pf % _DEPTH
            if out[pb] is not None:
                out[pb].wait()
                out[pb] = None
            gat[pb] = _start_gather(pf, pb)
        ht, hp = gat[bb]
        ht.wait()
        hp.wait()
        _accum(bb)
        out[bb] = pltpu.async_copy(
            toks[bb],
            out_hbm.at[pl.ds(brow * _S + o + ch * _G, _G), :],
            semo[bb])
    for h in out:
        if h is not None:
            h.wait()


def kernel(input_ids, embed_tokens, embed_positions):
    ids_flat = input_ids.reshape(-1)
    hidden, attn = _embed_kernel(ids_flat, embed_tokens, embed_positions)
    return hidden.reshape(_B, _S, _D), attn.reshape(_B, _S)


# speedup vs baseline: 1.5257x; 1.0220x over previous
import functools

import jax
import jax.numpy as jnp
from jax import lax
from jax.experimental import pallas as pl
from jax.experimental.pallas import tpu as pltpu
from jax.experimental.pallas import tpu_sc as plsc

_PAD = 1
_B, _S, _D = 4, 2048, 1024
_NC, _NS, _L = 2, 16, 16
_NW = _NC * _NS
_TPW = (_B * _S) // _NW
_WPR = _NW // _B
_G = 16
_NCH = _TPW // _G

_mesh = plsc.VectorSubcoreMesh(core_axis_name="c", subcore_axis_name="s")


@functools.partial(
    pl.kernel,
    out_type=(
        jax.ShapeDtypeStruct((_B, _S, _D), jnp.float32),
        jax.ShapeDtypeStruct((_B, _S), jnp.int32),
    ),
    mesh=_mesh,
    scratch_types=[
        pltpu.VMEM((_S,), jnp.int32),
        pltpu.VMEM((_TPW,), jnp.int32),
        pltpu.VMEM((_TPW,), jnp.int32),
        pltpu.VMEM((_G, _D), jnp.float32),
        pltpu.VMEM((_G, _D), jnp.float32),
        pltpu.VMEM((_G, _D), jnp.float32),
        pltpu.VMEM((_G, _D), jnp.float32),
        pltpu.VMEM((_G, _D), jnp.float32),
        pltpu.VMEM((_G, _D), jnp.float32),
        pltpu.SemaphoreType.DMA,
        pltpu.SemaphoreType.DMA,
        pltpu.SemaphoreType.DMA,
        pltpu.SemaphoreType.DMA,
        pltpu.SemaphoreType.DMA,
        pltpu.SemaphoreType.DMA,
        pltpu.SemaphoreType.DMA,
        pltpu.SemaphoreType.DMA,
        pltpu.SemaphoreType.DMA,
    ],
    compiler_params=pltpu.CompilerParams(needs_layout_passes=False),
)
def _embed_kernel(ids_hbm, tok_hbm, pos_hbm, out_hbm, attn_hbm,
                  row_v, pid_v, attn_v, tok0, tok1, tok2, pos0, pos1, pos2,
                  st0, st1, st2, sp0, sp1, sp2, so0, so1, so2):
    wid = lax.axis_index("s") * _NC + lax.axis_index("c")
    brow = wid // _WPR
    o = (wid % _WPR) * _TPW

    toks = (tok0, tok1, tok2)
    poss = (pos0, pos1, pos2)
    semt = (st0, st1, st2)
    semp = (sp0, sp1, sp2)
    semo = (so0, so1, so2)

    pltpu.sync_copy(ids_hbm.at[brow], row_v)

    def _pc_body(i, acc):
        ids16 = row_v[pl.ds(i * _L, _L)]
        return acc + jnp.where(ids16 != _PAD, 1, 0)

    acc = lax.fori_loop(0, o // _L, _pc_body, jnp.zeros((_L,), jnp.int32))
    carry0 = jnp.sum(acc)

    def _pos_body(k, carry):
        ids16 = row_v[pl.ds(o + k * _L, _L)]
        attn16 = jnp.where(ids16 != _PAD, 1, 0)
        cum = plsc.cumsum(attn16)
        pos16 = jnp.maximum(carry + cum - 1, 0)
        pid_v[pl.ds(k * _L, _L)] = pos16
        attn_v[pl.ds(k * _L, _L)] = attn16
        return carry + jnp.sum(attn16)

    lax.fori_loop(0, _TPW // _L, _pos_body, carry0)

    def _start_gather(ch, bb):
        cbase = ch * _G
        ht = pltpu.async_copy(
            tok_hbm.at[row_v.at[pl.ds(o + cbase, _G)]], toks[bb], semt[bb])
        hp = pltpu.async_copy(
            pos_hbm.at[pid_v.at[pl.ds(cbase, _G)]], poss[bb], semp[bb])
        return ht, hp

    def _accum(bb):
        tr, pr = toks[bb], poss[bb]

        @plsc.parallel_loop(0, _G)
        def _(r):
            for j in range(_D // _L):
                x = pr[r, pl.ds(j * _L, _L)]
                plsc.addupdate(tr.at[r, pl.ds(j * _L, _L)], x)

    _DEPTH = 3
    gat = [None] * _DEPTH
    out = [None] * _DEPTH
    for pf in range(_DEPTH - 1):
        gat[pf % _DEPTH] = _start_gather(pf, pf % _DEPTH)
    pltpu.sync_copy(attn_v, attn_hbm.at[brow, pl.ds(o, _TPW)])
    for ch in range(_NCH):
        bb = ch % _DEPTH
        pf = ch + _DEPTH - 1
        if pf < _NCH:
            pb = pf % _DEPTH
            if out[pb] is not None:
                out[pb].wait()
                out[pb] = None
            gat[pb] = _start_gather(pf, pb)
        ht, hp = gat[bb]
        ht.wait()
        hp.wait()
        _accum(bb)
        out[bb] = pltpu.async_copy(
            toks[bb],
            out_hbm.at[brow, pl.ds(o + ch * _G, _G), :],
            semo[bb])
    for h in out:
        if h is not None:
            h.wait()


def kernel(input_ids, embed_tokens, embed_positions):
    return _embed_kernel(input_ids, embed_tokens, embed_positions)
